# baseline (speedup 1.0000x reference)
import functools

import jax
import jax.numpy as jnp
from jax import lax
from jax.experimental import pallas as pl
from jax.experimental.pallas import tpu as pltpu
from jax.experimental.pallas import tpu_sc as plsc

_L = 16
_UNROLL = 8


def _sc_rcumsum(x_hbm, o_hbm, row_v, *, rows_per_worker, n):
    nchunks = n // _L
    nblocks = nchunks // _UNROLL
    wid = lax.axis_index("s") * 2 + lax.axis_index("c")

    def row_body(r, _):
        row = wid * rows_per_worker + r
        pltpu.sync_copy(x_hbm.at[row], row_v)

        def blk_body(b, carry):
            base = (nblocks - 1 - b) * _UNROLL
            vs, cums, tots = [], [], []
            for u in range(_UNROLL):
                v = row_v[pl.ds((base + u) * _L, _L)]
                vs.append(v)
                cums.append(plsc.cumsum(v))
                tots.append(jnp.sum(v))
            acc = carry
            for u in reversed(range(_UNROLL)):
                row_v[pl.ds((base + u) * _L, _L)] = (
                    (acc + tots[u]) - cums[u] + vs[u]
                )
                acc = acc + tots[u]
            return acc

        lax.fori_loop(0, nblocks, blk_body, jnp.float32(0.0))
        pltpu.sync_copy(row_v, o_hbm.at[row])
        return _

    lax.fori_loop(0, rows_per_worker, row_body, jnp.int32(0))


def kernel(x):
    m, n = x.shape
    info = plsc.get_sparse_core_info()
    nw = info.num_cores * info.num_subcores
    rows_per_worker = m // nw
    mesh = plsc.VectorSubcoreMesh(core_axis_name="c", subcore_axis_name="s")
    sc_k = functools.partial(
        pl.kernel,
        out_type=jax.ShapeDtypeStruct((m, n), x.dtype),
        mesh=mesh,
        scratch_types=[pltpu.VMEM((n,), jnp.float32)],
        compiler_params=pltpu.CompilerParams(needs_layout_passes=False),
    )(functools.partial(_sc_rcumsum, rows_per_worker=rows_per_worker, n=n))
    return sc_k(x)


# speedup vs baseline: 2.8972x; 2.8972x over previous
import functools

import jax
import jax.numpy as jnp
from jax import lax
from jax.experimental import pallas as pl
from jax.experimental.pallas import tpu as pltpu
from jax.experimental.pallas import tpu_sc as plsc

_L = 16
_UNROLL = 8
_G = 2


def _rcumsum_rows(in_ref, out_ref, n):
    nblocks = n // (_L * _UNROLL)
    lane15 = jnp.full((_L, 1), _L - 1, dtype=jnp.int32)
    bcast_last = functools.partial(
        lax.gather,
        dimension_numbers=lax.GatherDimensionNumbers(
            offset_dims=(), collapsed_slice_dims=(0,), start_index_map=(0,)),
        slice_sizes=(1,),
        mode=lax.GatherScatterMode.PROMISE_IN_BOUNDS,
    )

    for ri in range(_G):
        def blk_body(b, carry_vec):
            base = (nblocks - 1 - b) * _UNROLL
            vs, cums, tots = [], [], []
            for u in range(_UNROLL):
                v = in_ref[ri, pl.ds((base + u) * _L, _L)]
                c = plsc.cumsum(v)
                t = bcast_last(c, lane15)
                vs.append(v)
                cums.append(c)
                tots.append(t)
            psuf = [None] * _UNROLL
            acc = None
            for u in reversed(range(_UNROLL)):
                psuf[u] = acc
                acc = tots[u] if acc is None else acc + tots[u]
            for u in range(_UNROLL):
                local = tots[u] - cums[u] + vs[u]
                if psuf[u] is not None:
                    local = local + psuf[u]
                out_ref[ri, pl.ds((base + u) * _L, _L)] = carry_vec + local
            return carry_vec + acc

        lax.fori_loop(0, nblocks, blk_body, jnp.zeros((_L,), jnp.float32))


def _sc_rcumsum(x_hbm, o_hbm, in_a, in_b, out_a, out_b, sia, sib, soa, sob,
                *, rows_per_worker, n):
    wid = lax.axis_index("s") * 2 + lax.axis_index("c")
    row0 = wid * rows_per_worker
    ngroups = rows_per_worker // _G
    npairs = ngroups // 2

    def copy_in(buf, sem, g):
        return pltpu.make_async_copy(
            x_hbm.at[pl.ds(row0 + g * _G, _G)], buf, sem)

    def copy_out(buf, sem, g):
        return pltpu.make_async_copy(
            buf, o_hbm.at[pl.ds(row0 + g * _G, _G)], sem)

    copy_in(in_a, sia, 0).start()

    def pair_body(k, _):
        ga = 2 * k
        copy_in(in_a, sia, ga).wait()
        copy_in(in_b, sib, ga + 1).start()

        @pl.when(k > 0)
        def _():
            copy_out(out_a, soa, ga - 2).wait()

        _rcumsum_rows(in_a, out_a, n)
        copy_out(out_a, soa, ga).start()

        copy_in(in_b, sib, ga + 1).wait()

        @pl.when(k < npairs - 1)
        def _():
            copy_in(in_a, sia, ga + 2).start()

        @pl.when(k > 0)
        def _():
            copy_out(out_b, sob, ga - 1).wait()

        _rcumsum_rows(in_b, out_b, n)
        copy_out(out_b, sob, ga + 1).start()
        return jnp.int32(0)

    lax.fori_loop(0, npairs, pair_body, jnp.int32(0))
    copy_out(out_a, soa, ngroups - 2).wait()
    copy_out(out_b, sob, ngroups - 1).wait()


def kernel(x):
    m, n = x.shape
    info = plsc.get_sparse_core_info()
    nw = info.num_cores * info.num_subcores
    rows_per_worker = m // nw
    mesh = plsc.VectorSubcoreMesh(core_axis_name="c", subcore_axis_name="s")
    buf = pltpu.VMEM((_G, n), jnp.float32)
    sc_k = functools.partial(
        pl.kernel,
        out_type=jax.ShapeDtypeStruct((m, n), x.dtype),
        mesh=mesh,
        scratch_types=[buf, buf, buf, buf,
                       pltpu.SemaphoreType.DMA, pltpu.SemaphoreType.DMA,
                       pltpu.SemaphoreType.DMA, pltpu.SemaphoreType.DMA],
        compiler_params=pltpu.CompilerParams(needs_layout_passes=False),
    )(functools.partial(_sc_rcumsum, rows_per_worker=rows_per_worker, n=n))
    return sc_k(x)
